# baseline (speedup 1.0000x reference)
import jax
import jax.numpy as jnp
from jax.experimental import pallas as pl
from jax.experimental.pallas import tpu as pltpu

_B = 16384
_K = 8192
_D = 32
_TB = 4096
_SUB = 8
_KC = 512


def _assign_kernel(emb_ref, proto_ref, out_ref, pn_ref):
    i = pl.program_id(0)

    @pl.when(i == 0)
    def _():
        proto = proto_ref[...]
        pn_ref[...] = proto / jnp.clip(
            jnp.sqrt(jnp.sum(proto * proto, axis=1, keepdims=True)), 1e-12)

    emb = emb_ref[...]
    en = emb / jnp.clip(
        jnp.sqrt(jnp.sum(emb * emb, axis=1, keepdims=True)), 1e-12)

    acc_v = jnp.full((_SUB, _TB), -jnp.inf, jnp.float32)
    acc_i = jnp.zeros((_SUB, _TB), jnp.int32)
    for c in range(_K // _KC):
        simsT = jax.lax.dot_general(
            pn_ref[c * _KC:(c + 1) * _KC, :], en,
            (((1,), (1,)), ((), ())),
            preferred_element_type=jnp.float32)
        for rr in range(_KC // _SUB):
            r = c * (_KC // _SUB) + rr
            v = simsT[rr * _SUB:(rr + 1) * _SUB, :]
            m = v > acc_v
            acc_v = jnp.maximum(acc_v, v)
            acc_i = jnp.where(m, r, acc_i)

    sub = jax.lax.broadcasted_iota(jnp.int32, (_SUB, _TB), 0)
    g = acc_i * _SUB + sub
    colmax = jnp.max(acc_v, axis=0, keepdims=True)
    cand = jnp.where(acc_v == colmax, g, jnp.int32(2 ** 30))
    out_ref[0, :] = jnp.min(cand, axis=0)


def kernel(embeddings, prototypes):
    out = pl.pallas_call(
        _assign_kernel,
        grid=(_B // _TB,),
        in_specs=[
            pl.BlockSpec((_TB, _D), lambda i: (i, 0)),
            pl.BlockSpec((_K, _D), lambda i: (0, 0)),
        ],
        out_specs=pl.BlockSpec((1, _TB), lambda i: (0, i)),
        out_shape=jax.ShapeDtypeStruct((1, _B), jnp.int32),
        scratch_shapes=[pltpu.VMEM((_K, _D), jnp.float32)],
    )(embeddings, prototypes)
    return out[0]


# speedup vs baseline: 1.0086x; 1.0086x over previous
import jax
import jax.numpy as jnp
from jax.experimental import pallas as pl
from jax.experimental.pallas import tpu as pltpu

_B = 16384
_K = 8192
_D = 32
_TB = 2048
_SUB = 8
_KC = 1024


def _assign_kernel(emb_ref, proto_ref, out_ref, pn_ref):
    i = pl.program_id(0)

    @pl.when(i == 0)
    def _():
        proto = proto_ref[...]
        pn_ref[...] = proto / jnp.clip(
            jnp.sqrt(jnp.sum(proto * proto, axis=1, keepdims=True)), 1e-12)

    emb = emb_ref[...]
    en = emb / jnp.clip(
        jnp.sqrt(jnp.sum(emb * emb, axis=1, keepdims=True)), 1e-12)

    acc_v = jnp.full((_SUB, _TB), -jnp.inf, jnp.float32)
    acc_i = jnp.zeros((_SUB, _TB), jnp.int32)
    for c in range(_K // _KC):
        simsT = jax.lax.dot_general(
            pn_ref[c * _KC:(c + 1) * _KC, :], en,
            (((1,), (1,)), ((), ())),
            preferred_element_type=jnp.float32)
        for rr in range(_KC // _SUB):
            r = c * (_KC // _SUB) + rr
            v = simsT[rr * _SUB:(rr + 1) * _SUB, :]
            m = v > acc_v
            acc_v = jnp.maximum(acc_v, v)
            acc_i = jnp.where(m, r, acc_i)

    sub = jax.lax.broadcasted_iota(jnp.int32, (_SUB, _TB), 0)
    g = acc_i * _SUB + sub
    colmax = jnp.max(acc_v, axis=0, keepdims=True)
    cand = jnp.where(acc_v == colmax, g, jnp.int32(2 ** 30))
    out_ref[0, :] = jnp.min(cand, axis=0)


def kernel(embeddings, prototypes):
    out = pl.pallas_call(
        _assign_kernel,
        grid=(_B // _TB,),
        in_specs=[
            pl.BlockSpec((_TB, _D), lambda i: (i, 0)),
            pl.BlockSpec((_K, _D), lambda i: (0, 0)),
        ],
        out_specs=pl.BlockSpec((1, _TB), lambda i: (0, i)),
        out_shape=jax.ShapeDtypeStruct((1, _B), jnp.int32),
        scratch_shapes=[pltpu.VMEM((_K, _D), jnp.float32)],
    )(embeddings, prototypes)
    return out[0]


# speedup vs baseline: 1.0086x; 1.0000x over previous
import jax
import jax.numpy as jnp
from jax.experimental import pallas as pl
from jax.experimental.pallas import tpu as pltpu

_B = 16384
_K = 8192
_D = 32
_TB = 2048
_SUB = 8
_KC = 1024


def _assign_kernel(emb_ref, proto_ref, out_ref, pn_ref):
    i = pl.program_id(0)

    @pl.when(i == 0)
    def _():
        proto = proto_ref[...]
        pn_ref[...] = proto / jnp.clip(
            jnp.sqrt(jnp.sum(proto * proto, axis=1, keepdims=True)), 1e-12)

    emb = emb_ref[...]
    en = emb / jnp.clip(
        jnp.sqrt(jnp.sum(emb * emb, axis=1, keepdims=True)), 1e-12)

    acc_v = jnp.full((_SUB, _TB), -jnp.inf, jnp.float32)
    acc_i = jnp.zeros((_SUB, _TB), jnp.int32)
    for c in range(_K // _KC):
        simsT = jax.lax.dot_general(
            pn_ref[c * _KC:(c + 1) * _KC, :], en,
            (((1,), (1,)), ((), ())),
            preferred_element_type=jnp.float32)
        for rr in range(_KC // _SUB):
            r = c * (_KC // _SUB) + rr
            v = simsT[rr * _SUB:(rr + 1) * _SUB, :]
            m = v > acc_v
            acc_v = jnp.maximum(acc_v, v)
            acc_i = jnp.where(m, r, acc_i)

    sub = jax.lax.broadcasted_iota(jnp.int32, (_SUB, _TB), 0)
    g = acc_i * _SUB + sub
    colmax = jnp.max(acc_v, axis=0, keepdims=True)
    cand = jnp.where(acc_v == colmax, g, jnp.int32(2 ** 30))
    out_ref[...] = jnp.min(cand, axis=0)


def kernel(embeddings, prototypes):
    out = pl.pallas_call(
        _assign_kernel,
        grid=(_B // _TB,),
        in_specs=[
            pl.BlockSpec((_TB, _D), lambda i: (i, 0)),
            pl.BlockSpec((_K, _D), lambda i: (0, 0)),
        ],
        out_specs=pl.BlockSpec((_TB,), lambda i: (i,)),
        out_shape=jax.ShapeDtypeStruct((_B,), jnp.int32),
        scratch_shapes=[pltpu.VMEM((_K, _D), jnp.float32)],
    )(embeddings, prototypes)
    return out
